# baseline (speedup 1.0000x reference)
import functools

import jax
import jax.numpy as jnp
from jax.experimental import pallas as pl
from jax.experimental.pallas import tpu as pltpu

BIG = 1e10


def _sdtw_kernel(x_hbm, y_hbm, out_ref, s_ref, xbuf, ybuf, xsem, ysem,
                 *, bblk, n, d):
    nchunk = n // 128
    gi = pl.program_id(0)

    ones_row = jnp.ones((1, d), jnp.float32)

    def copy_in(b, slot):
        gb = gi * bblk + b
        pltpu.make_async_copy(x_hbm.at[gb], xbuf.at[slot], xsem.at[slot]).start()
        pltpu.make_async_copy(y_hbm.at[gb], ybuf.at[slot], ysem.at[slot]).start()

    copy_in(0, 0)

    def batch_body(b, carry):
        slot = jax.lax.rem(b, 2)

        @pl.when(b + 1 < bblk)
        def _():
            copy_in(b + 1, jax.lax.rem(b + 1, 2))

        pltpu.make_async_copy(xbuf.at[slot], xbuf.at[slot], xsem.at[slot]).wait()
        pltpu.make_async_copy(ybuf.at[slot], ybuf.at[slot], ysem.at[slot]).wait()
        xb = xbuf[slot]
        xnr = jax.lax.dot_general(
            ones_row, xb * xb, (((1,), (1,)), ((), ())),
            preferred_element_type=jnp.float32,
        )
        for qi in range(nchunk):
            yq = ybuf[slot, qi * 128:(qi + 1) * 128, :]
            yn = jnp.sum(yq * yq, axis=1, keepdims=True)
            g = jax.lax.dot_general(
                yq, xb, (((1,), (1,)), ((), ())),
                preferred_element_type=jnp.float32,
            )
            d2 = yn + xnr - 2.0 * g
            s_ref[b, qi * 128:(qi + 1) * 128, :] = jnp.sqrt(jnp.maximum(d2, 0.0))
        for pj in range(nchunk):
            blk = s_ref[b, :, pj * 128:(pj + 1) * 128]
            blk = pltpu.roll(blk, pj * 128, axis=0)
            lane = jax.lax.broadcasted_iota(jnp.int32, (n, 128), 1)
            for bit in range(7):
                sh = 1 << bit
                rolled = pltpu.roll(blk, sh, axis=0)
                blk = jnp.where((lane & sh) != 0, rolled, blk)
            s_ref[b, :, pj * 128:(pj + 1) * 128] = blk
        return carry

    jax.lax.fori_loop(0, bblk, batch_body, 0)

    big = jnp.float32(BIG)
    p_iota = jax.lax.broadcasted_iota(jnp.int32, (bblk, n), 1)

    d0 = s_ref[:, 0, :]
    r1 = jnp.where(p_iota == 0, d0, big)
    r2 = jnp.full((bblk, n), big, jnp.float32)

    def diag_body(s, carry):
        r1, r2 = carry
        c = jax.lax.bitwise_and(s, n - 1)
        dvals = s_ref[:, pl.ds(c, 1), :].reshape(bblk, n)
        up = pltpu.roll(r1, 1, axis=1)
        dg = pltpu.roll(r2, 1, axis=1)
        up = jnp.where(p_iota == 0, big, up)
        dg = jnp.where(p_iota == 0, big, dg)
        lf = r1
        m = jnp.minimum(jnp.minimum(up, dg), lf)
        ssum = (jnp.exp(m - up) + jnp.exp(m - dg) + jnp.exp(m - lf))
        r_new = dvals + m - jnp.log(ssum)
        valid = (p_iota <= s) & (p_iota > s - n)
        r_new = jnp.where(valid, r_new, big)
        return (r_new, r1)

    r1, r2 = jax.lax.fori_loop(1, 2 * n - 1, diag_body, (r1, r2))
    out_ref[...] = r1[:, n - 1:n]


@jax.jit
def kernel(X, Y):
    B, N, d = X.shape
    bblk = 32
    out = pl.pallas_call(
        functools.partial(_sdtw_kernel, bblk=bblk, n=N, d=d),
        grid=(B // bblk,),
        in_specs=[
            pl.BlockSpec(memory_space=pl.ANY),
            pl.BlockSpec(memory_space=pl.ANY),
        ],
        out_specs=pl.BlockSpec((bblk, 1), lambda i: (i, 0)),
        out_shape=jax.ShapeDtypeStruct((B, 1), jnp.float32),
        scratch_shapes=[
            pltpu.VMEM((bblk, N, N), jnp.float32),
            pltpu.VMEM((2, N, d), jnp.float32),
            pltpu.VMEM((2, N, d), jnp.float32),
            pltpu.SemaphoreType.DMA((2,)),
            pltpu.SemaphoreType.DMA((2,)),
        ],
        compiler_params=pltpu.CompilerParams(
            dimension_semantics=("arbitrary",),
            vmem_limit_bytes=40 * 1024 * 1024,
        ),
    )(X, Y)
    return out.reshape(B)


# speedup vs baseline: 1.0316x; 1.0316x over previous
import functools

import jax
import jax.numpy as jnp
from jax.experimental import pallas as pl
from jax.experimental.pallas import tpu as pltpu

BIG = 1e10


def _sdtw_kernel(x_hbm, y_hbm, out_ref, s_ref, xbuf, ybuf, xsem, ysem,
                 *, bblk, n, d):
    nchunk = n // 128
    gi = pl.program_id(0)

    ones_row = jnp.ones((1, d), jnp.float32)

    def copy_in(b, slot):
        gb = gi * bblk + b
        pltpu.make_async_copy(x_hbm.at[gb], xbuf.at[slot], xsem.at[slot]).start()
        pltpu.make_async_copy(y_hbm.at[gb], ybuf.at[slot], ysem.at[slot]).start()

    copy_in(0, 0)

    def batch_body(b, carry):
        slot = jax.lax.rem(b, 2)

        @pl.when(b + 1 < bblk)
        def _():
            copy_in(b + 1, jax.lax.rem(b + 1, 2))

        pltpu.make_async_copy(xbuf.at[slot], xbuf.at[slot], xsem.at[slot]).wait()
        pltpu.make_async_copy(ybuf.at[slot], ybuf.at[slot], ysem.at[slot]).wait()
        xb = xbuf[slot]
        xnr = jax.lax.dot_general(
            ones_row, xb * xb, (((1,), (1,)), ((), ())),
            preferred_element_type=jnp.float32,
        )
        for qi in range(nchunk):
            yq = ybuf[slot, qi * 128:(qi + 1) * 128, :]
            yn = jnp.sum(yq * yq, axis=1, keepdims=True)
            g = jax.lax.dot_general(
                yq, xb, (((1,), (1,)), ((), ())),
                preferred_element_type=jnp.float32,
            )
            d2 = yn + xnr - 2.0 * g
            s_ref[b, qi * 128:(qi + 1) * 128, :] = jnp.sqrt(jnp.maximum(d2, 0.0))
        for pj in range(nchunk):
            blk = s_ref[b, :, pj * 128:(pj + 1) * 128]
            blk = pltpu.roll(blk, pj * 128, axis=0)
            lane = jax.lax.broadcasted_iota(jnp.int32, (n, 128), 1)
            for bit in range(7):
                sh = 1 << bit
                rolled = pltpu.roll(blk, sh, axis=0)
                blk = jnp.where((lane & sh) != 0, rolled, blk)
            s_ref[b, :, pj * 128:(pj + 1) * 128] = blk
        return carry

    jax.lax.fori_loop(0, bblk, batch_body, 0)

    big = jnp.float32(BIG)
    half = bblk // 2
    p_iota = jax.lax.broadcasted_iota(jnp.int32, (half, n), 1)
    mask0 = p_iota == 0

    def step(dvals, r1, r2):
        up = pltpu.roll(r1, 1, axis=1)
        dg = pltpu.roll(r2, 1, axis=1)
        up = jnp.where(mask0, big, up)
        dg = jnp.where(mask0, big, dg)
        m = jnp.minimum(jnp.minimum(up, dg), r1)
        ssum = jnp.exp(m - up) + jnp.exp(m - dg) + jnp.exp(m - r1)
        return dvals + m - jnp.log(ssum)

    d0a = s_ref[0:half, 0, :]
    d0b = s_ref[half:bblk, 0, :]
    ra1 = jnp.where(mask0, d0a, big)
    rb1 = jnp.where(mask0, d0b, big)
    ra2 = jnp.full((half, n), big, jnp.float32)
    rb2 = jnp.full((half, n), big, jnp.float32)

    def diag_body(s, carry):
        ra1, ra2, rb1, rb2 = carry
        c = jax.lax.bitwise_and(s, n - 1)
        da = s_ref[0:half, pl.ds(c, 1), :].reshape(half, n)
        db = s_ref[half:bblk, pl.ds(c, 1), :].reshape(half, n)
        return (step(da, ra1, ra2), ra1, step(db, rb1, rb2), rb1)

    ra1, ra2, rb1, rb2 = jax.lax.fori_loop(
        1, 2 * n - 1, diag_body, (ra1, ra2, rb1, rb2))
    out_ref[0:half, :] = ra1[:, n - 1:n]
    out_ref[half:bblk, :] = rb1[:, n - 1:n]


@jax.jit
def kernel(X, Y):
    B, N, d = X.shape
    bblk = 32
    out = pl.pallas_call(
        functools.partial(_sdtw_kernel, bblk=bblk, n=N, d=d),
        grid=(B // bblk,),
        in_specs=[
            pl.BlockSpec(memory_space=pl.ANY),
            pl.BlockSpec(memory_space=pl.ANY),
        ],
        out_specs=pl.BlockSpec((bblk, 1), lambda i: (i, 0)),
        out_shape=jax.ShapeDtypeStruct((B, 1), jnp.float32),
        scratch_shapes=[
            pltpu.VMEM((bblk, N, N), jnp.float32),
            pltpu.VMEM((2, N, d), jnp.float32),
            pltpu.VMEM((2, N, d), jnp.float32),
            pltpu.SemaphoreType.DMA((2,)),
            pltpu.SemaphoreType.DMA((2,)),
        ],
        compiler_params=pltpu.CompilerParams(
            dimension_semantics=("arbitrary",),
            vmem_limit_bytes=40 * 1024 * 1024,
        ),
    )(X, Y)
    return out.reshape(B)


# speedup vs baseline: 1.1952x; 1.1585x over previous
import functools

import jax
import jax.numpy as jnp
from jax.experimental import pallas as pl
from jax.experimental.pallas import tpu as pltpu

BIG = 1e10


def _sdtw_kernel(x_hbm, y_hbm, out_ref, s_ref, t8_ref, xbuf, ybuf, xsem, ysem,
                 *, bblk, n, d):
    nchunk = n // 128
    gi = pl.program_id(0)

    ones_row = jnp.ones((1, d), jnp.float32)
    lane = jax.lax.broadcasted_iota(jnp.int32, (n, 128), 1)

    def copy_in(b, slot):
        gb = gi * bblk + b
        pltpu.make_async_copy(x_hbm.at[gb], xbuf.at[slot], xsem.at[slot]).start()
        pltpu.make_async_copy(y_hbm.at[gb], ybuf.at[slot], ysem.at[slot]).start()

    copy_in(0, 0)

    def group_body(g, carry):
        def batch_body(bi, carry):
            b = g * 8 + bi
            slot = jax.lax.rem(b, 2)

            @pl.when(b + 1 < bblk)
            def _():
                copy_in(b + 1, jax.lax.rem(b + 1, 2))

            pltpu.make_async_copy(xbuf.at[slot], xbuf.at[slot],
                                  xsem.at[slot]).wait()
            pltpu.make_async_copy(ybuf.at[slot], ybuf.at[slot],
                                  ysem.at[slot]).wait()
            xb = xbuf[slot]
            yb = ybuf[slot]
            xnr = jax.lax.dot_general(
                ones_row, xb * xb, (((1,), (1,)), ((), ())),
                preferred_element_type=jnp.float32,
            )
            yn = jnp.sum(yb * yb, axis=1, keepdims=True)
            for pj in range(nchunk):
                xj = xb[pj * 128:(pj + 1) * 128, :]
                gmm = jax.lax.dot_general(
                    yb, xj, (((1,), (1,)), ((), ())),
                    preferred_element_type=jnp.float32,
                )
                d2 = yn + xnr[:, pj * 128:(pj + 1) * 128] - 2.0 * gmm
                blk = jnp.sqrt(jnp.maximum(d2, 0.0))
                blk = pltpu.roll(blk, pj * 128, axis=0)
                for bit in range(7):
                    sh = 1 << bit
                    rolled = pltpu.roll(blk, sh, axis=0)
                    blk = jnp.where((lane & sh) != 0, rolled, blk)
                t8_ref[bi, :, pj * 128:(pj + 1) * 128] = blk
            return carry

        jax.lax.fori_loop(0, 8, batch_body, 0)

        def flush_body(ci, carry):
            c0 = ci * 8
            for pj in range(nchunk):
                t = t8_ref[:, pl.ds(c0, 8), pj * 128:(pj + 1) * 128]
                s_ref[pl.ds(c0, 8), pl.ds(g * 8, 8),
                      pj * 128:(pj + 1) * 128] = jnp.transpose(t, (1, 0, 2))
            return carry

        jax.lax.fori_loop(0, n // 8, flush_body, 0)
        return carry

    jax.lax.fori_loop(0, bblk // 8, group_body, 0)

    big = jnp.float32(BIG)
    half = bblk // 2
    p_iota = jax.lax.broadcasted_iota(jnp.int32, (half, n), 1)
    mask0 = p_iota == 0

    def step(dvals, r1, r2):
        up = pltpu.roll(r1, 1, axis=1)
        dg = pltpu.roll(r2, 1, axis=1)
        up = jnp.where(mask0, big, up)
        dg = jnp.where(mask0, big, dg)
        m = jnp.minimum(jnp.minimum(up, dg), r1)
        ssum = jnp.exp(m - up) + jnp.exp(m - dg) + jnp.exp(m - r1)
        return dvals + m - jnp.log(ssum)

    d0a = s_ref[0, 0:half, :]
    d0b = s_ref[0, half:bblk, :]
    ra1 = jnp.where(mask0, d0a, big)
    rb1 = jnp.where(mask0, d0b, big)
    ra2 = jnp.full((half, n), big, jnp.float32)
    rb2 = jnp.full((half, n), big, jnp.float32)

    def diag_body(s, carry):
        ra1, ra2, rb1, rb2 = carry
        c = jax.lax.bitwise_and(s, n - 1)
        da = s_ref[pl.ds(c, 1), 0:half, :].reshape(half, n)
        db = s_ref[pl.ds(c, 1), half:bblk, :].reshape(half, n)
        return (step(da, ra1, ra2), ra1, step(db, rb1, rb2), rb1)

    ra1, ra2, rb1, rb2 = jax.lax.fori_loop(
        1, 2 * n - 1, diag_body, (ra1, ra2, rb1, rb2))
    out_ref[0:half, :] = ra1[:, n - 1:n]
    out_ref[half:bblk, :] = rb1[:, n - 1:n]


@jax.jit
def kernel(X, Y):
    B, N, d = X.shape
    bblk = 32
    out = pl.pallas_call(
        functools.partial(_sdtw_kernel, bblk=bblk, n=N, d=d),
        grid=(B // bblk,),
        in_specs=[
            pl.BlockSpec(memory_space=pl.ANY),
            pl.BlockSpec(memory_space=pl.ANY),
        ],
        out_specs=pl.BlockSpec((bblk, 1), lambda i: (i, 0)),
        out_shape=jax.ShapeDtypeStruct((B, 1), jnp.float32),
        scratch_shapes=[
            pltpu.VMEM((N, bblk, N), jnp.float32),
            pltpu.VMEM((8, N, N), jnp.float32),
            pltpu.VMEM((2, N, d), jnp.float32),
            pltpu.VMEM((2, N, d), jnp.float32),
            pltpu.SemaphoreType.DMA((2,)),
            pltpu.SemaphoreType.DMA((2,)),
        ],
        compiler_params=pltpu.CompilerParams(
            dimension_semantics=("arbitrary",),
            vmem_limit_bytes=48 * 1024 * 1024,
        ),
    )(X, Y)
    return out.reshape(B)


# speedup vs baseline: 1.3003x; 1.0879x over previous
import functools

import jax
import jax.numpy as jnp
from jax.experimental import pallas as pl
from jax.experimental.pallas import tpu as pltpu

BIG = 1e10
LOG2E = 1.4426950408889634
LN2 = 0.6931471805599453


def _sdtw_kernel(x_hbm, y_hbm, out_ref, s_ref, t8_ref, xbuf, ybuf, xsem, ysem,
                 *, bblk, n, d):
    nchunk = n // 128
    gi = pl.program_id(0)

    ones_row = jnp.ones((1, d), jnp.float32)
    lane = jax.lax.broadcasted_iota(jnp.int32, (n, 128), 1)

    def copy_in(b, slot):
        gb = gi * bblk + b
        pltpu.make_async_copy(x_hbm.at[gb], xbuf.at[slot], xsem.at[slot]).start()
        pltpu.make_async_copy(y_hbm.at[gb], ybuf.at[slot], ysem.at[slot]).start()

    copy_in(0, 0)

    def group_body(g, carry):
        def batch_body(bi, carry):
            b = g * 8 + bi
            slot = jax.lax.rem(b, 2)

            @pl.when(b + 1 < bblk)
            def _():
                copy_in(b + 1, jax.lax.rem(b + 1, 2))

            pltpu.make_async_copy(xbuf.at[slot], xbuf.at[slot],
                                  xsem.at[slot]).wait()
            pltpu.make_async_copy(ybuf.at[slot], ybuf.at[slot],
                                  ysem.at[slot]).wait()
            xb = xbuf[slot]
            yb = ybuf[slot]
            xnr = jax.lax.dot_general(
                ones_row, xb * xb, (((1,), (1,)), ((), ())),
                preferred_element_type=jnp.float32,
            )
            yn = jnp.sum(yb * yb, axis=1, keepdims=True)
            for pj in range(nchunk):
                xj = xb[pj * 128:(pj + 1) * 128, :]
                gmm = jax.lax.dot_general(
                    yb, xj, (((1,), (1,)), ((), ())),
                    preferred_element_type=jnp.float32,
                )
                d2 = yn + xnr[:, pj * 128:(pj + 1) * 128] - 2.0 * gmm
                blk = jnp.sqrt(jnp.maximum(d2, 0.0)) * jnp.float32(LOG2E)
                blk = pltpu.roll(blk, pj * 128, axis=0)
                for bit in range(7):
                    sh = 1 << bit
                    rolled = pltpu.roll(blk, sh, axis=0)
                    blk = jnp.where((lane & sh) != 0, rolled, blk)
                t8_ref[bi, :, pj * 128:(pj + 1) * 128] = blk
            return carry

        jax.lax.fori_loop(0, 8, batch_body, 0)

        def flush_body(ci, carry):
            c0 = ci * 8
            for pj in range(nchunk):
                t = t8_ref[:, pl.ds(c0, 8), pj * 128:(pj + 1) * 128]
                s_ref[pl.ds(c0, 8), pl.ds(g * 8, 8),
                      pj * 128:(pj + 1) * 128] = jnp.transpose(t, (1, 0, 2))
            return carry

        jax.lax.fori_loop(0, n // 8, flush_body, 0)
        return carry

    jax.lax.fori_loop(0, bblk // 8, group_body, 0)

    big = jnp.float32(BIG)
    half = bblk // 2
    p_iota = jax.lax.broadcasted_iota(jnp.int32, (half, n), 1)
    mask0 = p_iota == 0

    def step(dvals, r1, r2):
        up = pltpu.roll(r1, 1, axis=1)
        dg = pltpu.roll(r2, 1, axis=1)
        m = jnp.minimum(jnp.minimum(up, dg), r1)
        ssum = jnp.exp2(m - up) + jnp.exp2(m - dg) + jnp.exp2(m - r1)
        return dvals + m - jnp.log2(ssum)

    def load(c, lo, hi):
        return s_ref[pl.ds(c, 1), lo:hi, :].reshape(hi - lo, n)

    d0a = s_ref[0, 0:half, :]
    d0b = s_ref[0, half:bblk, :]
    ra1 = jnp.where(mask0, d0a, big)
    rb1 = jnp.where(mask0, d0b, big)
    ra2 = jnp.full((half, n), big, jnp.float32)
    rb2 = jnp.full((half, n), big, jnp.float32)

    def diag_body(k, carry):
        ra1, ra2, rb1, rb2 = carry
        s1 = 2 * k + 1
        c1 = jax.lax.bitwise_and(s1, n - 1)
        c2 = jax.lax.bitwise_and(s1 + 1, n - 1)
        ra_n = step(load(c1, 0, half), ra1, ra2)
        rb_n = step(load(c1, half, bblk), rb1, rb2)
        ra_m = step(load(c2, 0, half), ra_n, ra1)
        rb_m = step(load(c2, half, bblk), rb_n, rb1)
        return (ra_m, ra_n, rb_m, rb_n)

    ra1, ra2, rb1, rb2 = jax.lax.fori_loop(
        0, n - 1, diag_body, (ra1, ra2, rb1, rb2))
    out_ref[0:half, :] = ra1[:, n - 1:n] * jnp.float32(LN2)
    out_ref[half:bblk, :] = rb1[:, n - 1:n] * jnp.float32(LN2)


@jax.jit
def kernel(X, Y):
    B, N, d = X.shape
    bblk = 32
    out = pl.pallas_call(
        functools.partial(_sdtw_kernel, bblk=bblk, n=N, d=d),
        grid=(B // bblk,),
        in_specs=[
            pl.BlockSpec(memory_space=pl.ANY),
            pl.BlockSpec(memory_space=pl.ANY),
        ],
        out_specs=pl.BlockSpec((bblk, 1), lambda i: (i, 0)),
        out_shape=jax.ShapeDtypeStruct((B, 1), jnp.float32),
        scratch_shapes=[
            pltpu.VMEM((N, bblk, N), jnp.float32),
            pltpu.VMEM((8, N, N), jnp.float32),
            pltpu.VMEM((2, N, d), jnp.float32),
            pltpu.VMEM((2, N, d), jnp.float32),
            pltpu.SemaphoreType.DMA((2,)),
            pltpu.SemaphoreType.DMA((2,)),
        ],
        compiler_params=pltpu.CompilerParams(
            dimension_semantics=("arbitrary",),
            vmem_limit_bytes=48 * 1024 * 1024,
        ),
    )(X, Y)
    return out.reshape(B)


# speedup vs baseline: 1.3783x; 1.0600x over previous
import functools

import jax
import jax.numpy as jnp
from jax.experimental import pallas as pl
from jax.experimental.pallas import tpu as pltpu

BIG = 1e10
LOG2E = 1.4426950408889634
LN2 = 0.6931471805599453


def _sdtw_kernel(x_hbm, y_hbm, out_ref, s_ref, t8_ref, xbuf, ybuf, xsem, ysem,
                 *, bblk, n, d):
    nchunk = n // 128
    gi = pl.program_id(0)

    ones_row = jnp.ones((1, d), jnp.float32)
    lane = jax.lax.broadcasted_iota(jnp.int32, (n, 128), 1)

    def copy_in(b, slot):
        gb = gi * bblk + b
        pltpu.make_async_copy(x_hbm.at[gb], xbuf.at[slot], xsem.at[slot]).start()
        pltpu.make_async_copy(y_hbm.at[gb], ybuf.at[slot], ysem.at[slot]).start()

    copy_in(0, 0)

    def group_body(g, carry):
        def batch_body(bi, carry):
            b = g * 8 + bi
            slot = jax.lax.rem(b, 2)

            @pl.when(b + 1 < bblk)
            def _():
                copy_in(b + 1, jax.lax.rem(b + 1, 2))

            pltpu.make_async_copy(xbuf.at[slot], xbuf.at[slot],
                                  xsem.at[slot]).wait()
            pltpu.make_async_copy(ybuf.at[slot], ybuf.at[slot],
                                  ysem.at[slot]).wait()
            xb = xbuf[slot]
            yb = ybuf[slot]
            xnr = jax.lax.dot_general(
                ones_row, xb * xb, (((1,), (1,)), ((), ())),
                preferred_element_type=jnp.float32,
            )
            yn = jnp.sum(yb * yb, axis=1, keepdims=True)
            for pj in range(nchunk):
                xj = xb[pj * 128:(pj + 1) * 128, :]
                gmm = jax.lax.dot_general(
                    yb, xj, (((1,), (1,)), ((), ())),
                    preferred_element_type=jnp.float32,
                )
                d2 = yn + xnr[:, pj * 128:(pj + 1) * 128] - 2.0 * gmm
                blk = jnp.sqrt(jnp.maximum(d2, 0.0)) * jnp.float32(LOG2E)
                blk = pltpu.roll(blk, pj * 128, axis=0)
                for bit in range(7):
                    sh = 1 << bit
                    rolled = pltpu.roll(blk, sh, axis=0)
                    blk = jnp.where((lane & sh) != 0, rolled, blk)
                t8_ref[bi, :, pj * 128:(pj + 1) * 128] = blk
            return carry

        jax.lax.fori_loop(0, 8, batch_body, 0)

        def flush_body(ci, carry):
            c0 = ci * 8
            for pj in range(nchunk):
                t = t8_ref[:, pl.ds(c0, 8), pj * 128:(pj + 1) * 128]
                s_ref[pl.ds(c0, 8), pl.ds(g * 8, 8),
                      pj * 128:(pj + 1) * 128] = jnp.transpose(t, (1, 0, 2))
            return carry

        jax.lax.fori_loop(0, n // 8, flush_body, 0)
        return carry

    jax.lax.fori_loop(0, bblk // 8, group_body, 0)

    big = jnp.float32(BIG)
    ng = 4
    qb = bblk // ng
    p_iota = jax.lax.broadcasted_iota(jnp.int32, (qb, n), 1)
    mask0 = p_iota == 0

    def step(dvals, r1, r2):
        up = pltpu.roll(r1, 1, axis=1)
        dg = pltpu.roll(r2, 1, axis=1)
        m = jnp.minimum(jnp.minimum(up, dg), r1)
        ssum = jnp.exp2(m - up) + jnp.exp2(m - dg) + jnp.exp2(m - r1)
        return dvals + m - jnp.log2(ssum)

    def load(c, qi):
        return s_ref[pl.ds(c, 1), qi * qb:(qi + 1) * qb, :].reshape(qb, n)

    r1s = [jnp.where(mask0, s_ref[0, qi * qb:(qi + 1) * qb, :], big)
           for qi in range(ng)]
    r2s = [jnp.full((qb, n), big, jnp.float32) for _ in range(ng)]

    def sweep(s, r1s, r2s):
        c = jax.lax.bitwise_and(s, n - 1)
        for qi in range(ng):
            new = step(load(c, qi), r1s[qi], r2s[qi])
            r2s[qi] = r1s[qi]
            r1s[qi] = new
        return r1s, r2s

    def diag_body(k, carry):
        r1s = list(carry[:ng])
        r2s = list(carry[ng:])
        s1 = 4 * k + 1
        for t in range(4):
            r1s, r2s = sweep(s1 + t, r1s, r2s)
        return tuple(r1s) + tuple(r2s)

    carry = jax.lax.fori_loop(
        0, (2 * n - 2) // 4, diag_body, tuple(r1s) + tuple(r2s))
    r1s = list(carry[:ng])
    r2s = list(carry[ng:])
    for s_tail in range(2 * n - 1 - 2, 2 * n - 1):
        r1s, r2s = sweep(jnp.int32(s_tail), r1s, r2s)
    for qi in range(ng):
        out_ref[qi * qb:(qi + 1) * qb, :] = (
            r1s[qi][:, n - 1:n] * jnp.float32(LN2))


@jax.jit
def kernel(X, Y):
    B, N, d = X.shape
    bblk = 32
    out = pl.pallas_call(
        functools.partial(_sdtw_kernel, bblk=bblk, n=N, d=d),
        grid=(B // bblk,),
        in_specs=[
            pl.BlockSpec(memory_space=pl.ANY),
            pl.BlockSpec(memory_space=pl.ANY),
        ],
        out_specs=pl.BlockSpec((bblk, 1), lambda i: (i, 0)),
        out_shape=jax.ShapeDtypeStruct((B, 1), jnp.float32),
        scratch_shapes=[
            pltpu.VMEM((N, bblk, N), jnp.float32),
            pltpu.VMEM((8, N, N), jnp.float32),
            pltpu.VMEM((2, N, d), jnp.float32),
            pltpu.VMEM((2, N, d), jnp.float32),
            pltpu.SemaphoreType.DMA((2,)),
            pltpu.SemaphoreType.DMA((2,)),
        ],
        compiler_params=pltpu.CompilerParams(
            dimension_semantics=("arbitrary",),
            vmem_limit_bytes=48 * 1024 * 1024,
        ),
    )(X, Y)
    return out.reshape(B)


# speedup vs baseline: 1.4070x; 1.0208x over previous
import functools

import jax
import jax.numpy as jnp
from jax.experimental import pallas as pl
from jax.experimental.pallas import tpu as pltpu

BIG = 1e10
LOG2E = 1.4426950408889634
LN2 = 0.6931471805599453


def _sdtw_kernel(x_hbm, y_hbm, out_ref, s_ref, t8_ref, xbuf, ybuf, xsem, ysem,
                 *, bblk, n, d):
    nchunk = n // 128
    gi = pl.program_id(0)

    ones_row = jnp.ones((1, d), jnp.float32)
    lane = jax.lax.broadcasted_iota(jnp.int32, (n, 128), 1)

    def copy_in(b, slot):
        gb = gi * bblk + b
        pltpu.make_async_copy(x_hbm.at[gb], xbuf.at[slot], xsem.at[slot]).start()
        pltpu.make_async_copy(y_hbm.at[gb], ybuf.at[slot], ysem.at[slot]).start()

    copy_in(0, 0)

    def group_body(g, carry):
        def batch_body(bi, carry):
            b = g * 8 + bi
            slot = jax.lax.rem(b, 2)

            @pl.when(b + 1 < bblk)
            def _():
                copy_in(b + 1, jax.lax.rem(b + 1, 2))

            pltpu.make_async_copy(xbuf.at[slot], xbuf.at[slot],
                                  xsem.at[slot]).wait()
            pltpu.make_async_copy(ybuf.at[slot], ybuf.at[slot],
                                  ysem.at[slot]).wait()
            xb = xbuf[slot]
            yb = ybuf[slot]
            xnr = jax.lax.dot_general(
                ones_row, xb * xb, (((1,), (1,)), ((), ())),
                preferred_element_type=jnp.float32,
            )
            yn = jnp.sum(yb * yb, axis=1, keepdims=True)
            for pj in range(nchunk):
                xj = xb[pj * 128:(pj + 1) * 128, :]
                gmm = jax.lax.dot_general(
                    yb, xj, (((1,), (1,)), ((), ())),
                    preferred_element_type=jnp.float32,
                )
                d2 = yn + xnr[:, pj * 128:(pj + 1) * 128] - 2.0 * gmm
                blk = jnp.sqrt(jnp.maximum(d2, 0.0)) * jnp.float32(LOG2E)
                blk = pltpu.roll(blk, pj * 128, axis=0)
                for bit in range(7):
                    sh = 1 << bit
                    rolled = pltpu.roll(blk, sh, axis=0)
                    blk = jnp.where((lane & sh) != 0, rolled, blk)
                t8_ref[bi, :, pj * 128:(pj + 1) * 128] = blk
            return carry

        jax.lax.fori_loop(0, 8, batch_body, 0)

        def flush_body(ci, carry):
            c0 = ci * 8
            for pj in range(nchunk):
                t = t8_ref[:, pl.ds(c0, 8), pj * 128:(pj + 1) * 128]
                s_ref[pl.ds(c0, 8), pl.ds(g * 8, 8),
                      pj * 128:(pj + 1) * 128] = jnp.transpose(t, (1, 0, 2))
            return carry

        jax.lax.fori_loop(0, n // 8, flush_body, 0)
        return carry

    jax.lax.fori_loop(0, bblk // 8, group_body, 0)

    big = jnp.float32(BIG)
    ng = 4
    qb = bblk // ng
    p_iota = jax.lax.broadcasted_iota(jnp.int32, (qb, n), 1)
    mask0 = p_iota == 0

    def softmin3(up, dg, lf):
        m = jnp.minimum(jnp.minimum(up, dg), lf)
        ssum = jnp.exp2(m - up) + jnp.exp2(m - dg) + jnp.exp2(m - lf)
        return m - jnp.log2(ssum)

    def step(dvals, r1, r2):
        up = pltpu.roll(r1, 1, axis=1)
        dg = pltpu.roll(r2, 1, axis=1)
        return dvals + softmin3(up, dg, r1)

    def load(c, qi):
        return s_ref[pl.ds(c, 1), qi * qb:(qi + 1) * qb, :].reshape(qb, n)

    r1s = [jnp.where(mask0, s_ref[0, qi * qb:(qi + 1) * qb, :], big)
           for qi in range(ng)]
    r2s = [jnp.full((qb, n), big, jnp.float32) for _ in range(ng)]

    def sweep(s, r1s, r2s):
        c = jax.lax.bitwise_and(s, n - 1)
        for qi in range(ng):
            new = step(load(c, qi), r1s[qi], r2s[qi])
            r2s[qi] = r1s[qi]
            r1s[qi] = new
        return r1s, r2s

    def pair_sweep(s, r1s, r2s):
        ca = jax.lax.bitwise_and(s, n - 1)
        cb = jax.lax.bitwise_and(s + 1, n - 1)
        for qi in range(ng):
            r1, r2 = r1s[qi], r2s[qi]
            u1 = pltpu.roll(r1, 1, axis=1)
            u2 = pltpu.roll(r1, 2, axis=1)
            v1 = pltpu.roll(r2, 1, axis=1)
            v2 = pltpu.roll(r2, 2, axis=1)
            da = load(ca, qi)
            db = load(cb, qi)
            dash = pltpu.roll(da, 1, axis=1)
            a = da + softmin3(u1, v1, r1)
            ash = dash + softmin3(u2, v2, u1)
            bb = db + softmin3(ash, u1, a)
            r1s[qi] = bb
            r2s[qi] = a
        return r1s, r2s

    def diag_body(k, carry):
        r1s = list(carry[:ng])
        r2s = list(carry[ng:])
        s1 = 4 * k + 1
        r1s, r2s = pair_sweep(s1, r1s, r2s)
        r1s, r2s = pair_sweep(s1 + 2, r1s, r2s)
        return tuple(r1s) + tuple(r2s)

    carry = jax.lax.fori_loop(
        0, (2 * n - 2) // 4, diag_body, tuple(r1s) + tuple(r2s))
    r1s = list(carry[:ng])
    r2s = list(carry[ng:])
    for s_tail in range(2 * n - 1 - 2, 2 * n - 1):
        r1s, r2s = sweep(jnp.int32(s_tail), r1s, r2s)
    for qi in range(ng):
        out_ref[qi * qb:(qi + 1) * qb, :] = (
            r1s[qi][:, n - 1:n] * jnp.float32(LN2))


@jax.jit
def kernel(X, Y):
    B, N, d = X.shape
    bblk = 32
    out = pl.pallas_call(
        functools.partial(_sdtw_kernel, bblk=bblk, n=N, d=d),
        grid=(B // bblk,),
        in_specs=[
            pl.BlockSpec(memory_space=pl.ANY),
            pl.BlockSpec(memory_space=pl.ANY),
        ],
        out_specs=pl.BlockSpec((bblk, 1), lambda i: (i, 0)),
        out_shape=jax.ShapeDtypeStruct((B, 1), jnp.float32),
        scratch_shapes=[
            pltpu.VMEM((N, bblk, N), jnp.float32),
            pltpu.VMEM((8, N, N), jnp.float32),
            pltpu.VMEM((2, N, d), jnp.float32),
            pltpu.VMEM((2, N, d), jnp.float32),
            pltpu.SemaphoreType.DMA((2,)),
            pltpu.SemaphoreType.DMA((2,)),
        ],
        compiler_params=pltpu.CompilerParams(
            dimension_semantics=("arbitrary",),
            vmem_limit_bytes=48 * 1024 * 1024,
        ),
    )(X, Y)
    return out.reshape(B)
